# baseline (speedup 1.0000x reference)
import functools

import jax
import jax.numpy as jnp
from jax import lax
from jax.experimental import pallas as pl
from jax.experimental.pallas import tpu as pltpu
from jax.experimental.pallas import tpu_sc as plsc

_ROWS, _D = 16384, 256
_NC, _NS, _L = 2, 16, 16
_NW = _NC * _NS
_RPW = _ROWS // _NW
_C = 64
_NCH = _RPW // _C
_CSH = 6
_BIG = 1 << 30

_mesh = plsc.VectorSubcoreMesh(core_axis_name="c", subcore_axis_name="s")


@functools.partial(
    pl.kernel,
    out_type=jax.ShapeDtypeStruct((_ROWS, _D), jnp.float32),
    mesh=_mesh,
    scratch_types=[
        pltpu.VMEM((_RPW,), jnp.int32),
        pltpu.VMEM((_NCH, _C), jnp.int32),
        pltpu.VMEM((_NCH, _C), jnp.int32),
        pltpu.VMEM((_NCH, _C), jnp.int32),
        pltpu.VMEM((_C, _D), jnp.float32),
        pltpu.VMEM((_C, _D), jnp.float32),
        pltpu.VMEM((_C,), jnp.int32),
        pltpu.SemaphoreType.DMA,
    ],
    compiler_params=pltpu.CompilerParams(needs_layout_passes=False),
)
def _sc_select(base_hbm, mask_hbm, oov_hbm, out_hbm,
               mask_v, idx_g, idx_si, idx_so, buf, oovbuf, idxz, sem):
    wid = lax.axis_index("s") * _NC + lax.axis_index("c")
    row0 = wid * _RPW

    pltpu.sync_copy(mask_hbm.at[pl.ds(row0, _RPW)], mask_v)

    zv = jnp.zeros((_L,), jnp.int32)
    for c in range(_C // _L):
        idxz[pl.ds(c * _L, _L)] = zv
    pltpu.async_copy(oov_hbm.at[idxz], oovbuf, sem).wait()

    big_v = jnp.full((_L,), _BIG, jnp.int32)
    cmask = jnp.full((_L,), _C - 1, jnp.int32)
    shv = jnp.full((_L,), _CSH, jnp.int32)
    off_in = jnp.zeros((_L,), jnp.int32)
    off_oov = jnp.zeros((_L,), jnp.int32)
    first_in = big_v
    first_oov = big_v
    row0_v = jnp.broadcast_to(row0, (_L,)).astype(jnp.int32)
    for v in range(_RPW // _L):
        mi = mask_v[pl.ds(v * _L, _L)]
        m = mi != jnp.zeros((_L,), jnp.int32)
        mn = jnp.logical_not(m)
        mni = mn.astype(jnp.int32)
        rows = lax.iota(jnp.int32, _L) + row0_v + jnp.full((_L,), v * _L, jnp.int32)
        pos_in = off_in + plsc.cumsum(mi) - mi
        pos_oov = off_oov + plsc.cumsum(mni) - mni
        plsc.store_scatter(idx_g, [pos_in >> shv, pos_in & cmask],
                           rows, mask=m)
        plsc.store_scatter(idx_si, [pos_in >> shv, pos_in & cmask],
                           rows, mask=m)
        plsc.store_scatter(idx_so, [pos_oov >> shv, pos_oov & cmask],
                           rows, mask=mn)
        off_in = off_in + plsc.all_reduce_population_count(m)
        off_oov = off_oov + plsc.all_reduce_population_count(mn)
        first_in = jnp.minimum(first_in, jnp.where(m, rows, big_v))
        first_oov = jnp.minimum(first_oov, jnp.where(mn, rows, big_v))

    n_in = jnp.max(off_in)
    n_oov = jnp.max(off_oov)
    nc_in = (n_in + _C - 1) >> _CSH
    nc_oov = (n_oov + _C - 1) >> _CSH

    lane = lax.iota(jnp.int32, _L)
    n_in_v = jnp.broadcast_to(n_in, (_L,))
    n_oov_v = jnp.broadcast_to(n_oov, (_L,))
    end_in_v = jnp.broadcast_to(nc_in << _CSH, (_L,))
    end_oov_v = jnp.broadcast_to(nc_oov << _CSH, (_L,))
    fv = jnp.broadcast_to(jnp.min(first_in), (_L,))
    fvo = jnp.broadcast_to(jnp.min(first_oov), (_L,))
    for j in range(_C // _L):
        jl = jnp.full((_L,), j * _L, jnp.int32)
        p_in = n_in_v + lane + jl
        tm = p_in < end_in_v
        plsc.store_scatter(idx_g, [p_in >> shv, p_in & cmask], fv, mask=tm)
        plsc.store_scatter(idx_si, [p_in >> shv, p_in & cmask], fv, mask=tm)
        p_o = n_oov_v + lane + jl
        tmo = p_o < end_oov_v
        plsc.store_scatter(idx_so, [p_o >> shv, p_o & cmask], fvo, mask=tmo)

    def body_in(t, carry):
        pltpu.async_copy(base_hbm.at[idx_g.at[t]], buf, sem).wait()
        pltpu.async_copy(buf, out_hbm.at[idx_si.at[t]], sem).wait()
        return carry
    lax.fori_loop(0, nc_in, body_in, 0)

    def body_oov(t, carry):
        pltpu.async_copy(oovbuf, out_hbm.at[idx_so.at[t]], sem).wait()
        return carry
    lax.fori_loop(0, nc_oov, body_oov, 0)


def kernel(base_embedding, in_vocab, oov_embedding):
    base = base_embedding.astype(jnp.float32)
    mask = in_vocab.astype(jnp.int32)
    oov2 = oov_embedding.astype(jnp.float32).reshape(1, _D)
    return _sc_select(base, mask, oov2)


# speedup vs baseline: 2.5877x; 2.5877x over previous
import functools

import jax
import jax.numpy as jnp
from jax import lax
from jax.experimental import pallas as pl
from jax.experimental.pallas import tpu as pltpu
from jax.experimental.pallas import tpu_sc as plsc

_ROWS, _D = 16384, 256
_NC, _NS, _L = 2, 16, 16
_NW = _NC * _NS
_RPW = _ROWS // _NW
_C = 64
_NCH = _RPW // _C
_CSH = 6
_NB = 4
_BIG = 1 << 30

_mesh = plsc.VectorSubcoreMesh(core_axis_name="c", subcore_axis_name="s")


@functools.partial(
    pl.kernel,
    out_type=jax.ShapeDtypeStruct((_ROWS, _D), jnp.float32),
    mesh=_mesh,
    scratch_types=[
        pltpu.VMEM((_RPW,), jnp.int32),
        pltpu.VMEM((_NCH, _C), jnp.int32),
        pltpu.VMEM((_NCH, _C), jnp.int32),
        pltpu.VMEM((_NCH, _C), jnp.int32),
        [pltpu.VMEM((_C, _D), jnp.float32) for _ in range(_NB)],
        pltpu.VMEM((_C, _D), jnp.float32),
        [pltpu.SemaphoreType.DMA for _ in range(_NB)],
        [pltpu.SemaphoreType.DMA for _ in range(_NB)],
        pltpu.SemaphoreType.DMA,
    ],
    compiler_params=pltpu.CompilerParams(needs_layout_passes=False),
)
def _sc_select(base_hbm, mask_hbm, oov_hbm, out_hbm,
               mask_v, idx_g, idx_si, idx_so, bufs, oovbuf,
               gsems, ssems, osem):
    wid = lax.axis_index("s") * _NC + lax.axis_index("c")
    row0 = wid * _RPW

    pltpu.sync_copy(mask_hbm.at[pl.ds(row0, _RPW)], mask_v)

    pltpu.sync_copy(oov_hbm, oovbuf.at[0])
    vecs = [oovbuf[0, pl.ds(j * _L, _L)] for j in range(_D // _L)]
    for r in range(1, _C):
        for j in range(_D // _L):
            oovbuf[r, pl.ds(j * _L, _L)] = vecs[j]

    big_v = jnp.full((_L,), _BIG, jnp.int32)
    cmask = jnp.full((_L,), _C - 1, jnp.int32)
    shv = jnp.full((_L,), _CSH, jnp.int32)
    zv = jnp.zeros((_L,), jnp.int32)
    off_in = zv
    off_oov = zv
    first_in = big_v
    first_oov = big_v
    row0_v = jnp.broadcast_to(row0, (_L,)).astype(jnp.int32)
    for v in range(_RPW // _L):
        mi = mask_v[pl.ds(v * _L, _L)]
        m = mi != zv
        mn = jnp.logical_not(m)
        mni = mn.astype(jnp.int32)
        rows = lax.iota(jnp.int32, _L) + row0_v + jnp.full((_L,), v * _L, jnp.int32)
        pos_in = off_in + plsc.cumsum(mi) - mi
        pos_oov = off_oov + plsc.cumsum(mni) - mni
        plsc.store_scatter(idx_g, [pos_in >> shv, pos_in & cmask],
                           rows, mask=m)
        plsc.store_scatter(idx_si, [pos_in >> shv, pos_in & cmask],
                           rows, mask=m)
        plsc.store_scatter(idx_so, [pos_oov >> shv, pos_oov & cmask],
                           rows, mask=mn)
        off_in = off_in + plsc.all_reduce_population_count(m)
        off_oov = off_oov + plsc.all_reduce_population_count(mn)
        first_in = jnp.minimum(first_in, jnp.where(m, rows, big_v))
        first_oov = jnp.minimum(first_oov, jnp.where(mn, rows, big_v))

    n_in = jnp.max(off_in)
    n_oov = jnp.max(off_oov)
    nc_in = (n_in + _C - 1) >> _CSH
    nc_oov = (n_oov + _C - 1) >> _CSH

    lane = lax.iota(jnp.int32, _L)
    n_in_v = jnp.broadcast_to(n_in, (_L,))
    n_oov_v = jnp.broadcast_to(n_oov, (_L,))
    end_in_v = jnp.broadcast_to(nc_in << _CSH, (_L,))
    end_oov_v = jnp.broadcast_to(nc_oov << _CSH, (_L,))
    fv = jnp.broadcast_to(jnp.min(first_in), (_L,))
    fvo = jnp.broadcast_to(jnp.min(first_oov), (_L,))
    for j in range(_C // _L):
        jl = jnp.full((_L,), j * _L, jnp.int32)
        p_in = n_in_v + lane + jl
        tm = p_in < end_in_v
        plsc.store_scatter(idx_g, [p_in >> shv, p_in & cmask], fv, mask=tm)
        plsc.store_scatter(idx_si, [p_in >> shv, p_in & cmask], fv, mask=tm)
        p_o = n_oov_v + lane + jl
        tmo = p_o < end_oov_v
        plsc.store_scatter(idx_so, [p_o >> shv, p_o & cmask], fvo, mask=tmo)

    def fire_oov(t):
        pltpu.async_copy(oovbuf, out_hbm.at[idx_so.at[t]], osem)

    for t in range(_NCH):
        pl.when(t < nc_oov)(functools.partial(fire_oov, t))

    def fire_gather(t):
        b = t % _NB
        if t >= _NB:
            pltpu.make_async_copy(bufs[b], out_hbm.at[idx_si.at[t - _NB]],
                                  ssems[b]).wait()
        pltpu.async_copy(base_hbm.at[idx_g.at[t]], bufs[b], gsems[b])

    def fire_scatter(t):
        b = t % _NB
        pltpu.make_async_copy(base_hbm.at[idx_g.at[t]], bufs[b],
                              gsems[b]).wait()
        pltpu.async_copy(bufs[b], out_hbm.at[idx_si.at[t]], ssems[b])

    for t in range(_NCH):
        pl.when(t < nc_in)(functools.partial(fire_gather, t))
        if t >= 1:
            pl.when((t - 1) < nc_in)(functools.partial(fire_scatter, t - 1))
    pl.when((_NCH - 1) < nc_in)(functools.partial(fire_scatter, _NCH - 1))

    def drain_scatter(t):
        b = t % _NB
        pltpu.make_async_copy(bufs[b], out_hbm.at[idx_si.at[t]],
                              ssems[b]).wait()

    for t in range(_NCH):
        pl.when((t < nc_in) & (t + _NB >= nc_in))(
            functools.partial(drain_scatter, t))

    def drain_oov(t):
        pltpu.make_async_copy(oovbuf, out_hbm.at[idx_so.at[t]], osem).wait()

    for t in range(_NCH):
        pl.when(t < nc_oov)(functools.partial(drain_oov, t))


def kernel(base_embedding, in_vocab, oov_embedding):
    base = base_embedding.astype(jnp.float32)
    mask = in_vocab.astype(jnp.int32)
    return _sc_select(base, mask, oov_embedding.astype(jnp.float32))
